# baseline (speedup 1.0000x reference)
import functools

import jax
import jax.numpy as jnp
from jax.experimental import pallas as pl
from jax.experimental.pallas import tpu as pltpu

E = 8
R = 32
D = 2048
O = 2048
T = 8192
ER = E * R

NSUB = 4


def _fused_kernel(x_ref, wcat_ref, rw_ref, tw_ref, bws_ref, rb_ref, tb_ref,
                  out_ref, bcat_ref):
    @pl.when(pl.program_id(0) == 0)
    def _prep():
        for e in range(E):
            bcat_ref[e * R:(e + 1) * R, :] = jnp.transpose(
                bws_ref[e * O:(e + 1) * O, :].astype(jnp.bfloat16))

    col_e = jax.lax.broadcasted_iota(jnp.int32, (E, ER), 1) // R
    row_e = jax.lax.broadcasted_iota(jnp.int32, (E, ER), 0)
    expand = jnp.where(col_e == row_e, 1.0, 0.0).astype(jnp.bfloat16)
    sub = x_ref.shape[0] // NSUB
    for i in range(NSUB):
        x = x_ref[i * sub:(i + 1) * sub, :].astype(jnp.bfloat16)
        l_t = jax.lax.dot_general(
            rw_ref[...].astype(jnp.bfloat16), x, (((1,), (1,)), ((), ())),
            preferred_element_type=jnp.float32)
        t_t = jax.lax.dot_general(
            tw_ref[...].astype(jnp.bfloat16), x, (((1,), (1,)), ((), ())),
            preferred_element_type=jnp.float32)
        u = jnp.exp(l_t + rb_ref[...])
        thr = jax.nn.sigmoid(t_t + tb_ref[...]) * (1.0 / E)
        v = jnp.maximum(u - thr * jnp.sum(u, axis=0, keepdims=True), 0.0)
        s = jnp.sum(v, axis=0, keepdims=True)
        s = jnp.where(s == 0.0, 1.0, s)
        w_t = (v / s).astype(jnp.bfloat16)
        h = jax.lax.dot_general(
            x, wcat_ref[...], (((1,), (1,)), ((), ())),
            preferred_element_type=jnp.float32)
        w_exp = jax.lax.dot_general(
            w_t, expand, (((0,), (0,)), ((), ())),
            preferred_element_type=jnp.float32).astype(jnp.bfloat16)
        hw = h.astype(jnp.bfloat16) * w_exp
        out_ref[i * sub:(i + 1) * sub, :] = jnp.dot(
            hw, bcat_ref[...], preferred_element_type=jnp.float32)


@functools.partial(jax.jit, static_argnames=("block_t",))
def _run(inputs, router_w, router_b, thr_w, thr_b, A_ws, B_ws, block_t=1024):
    w_cat = A_ws.reshape(ER, D).astype(jnp.bfloat16)
    b_flat = B_ws.reshape(E * O, R)
    rb = router_b.reshape(E, 1)
    tb = thr_b.reshape(1, 1)

    grid = (T // block_t,)
    return pl.pallas_call(
        _fused_kernel,
        grid=grid,
        in_specs=[
            pl.BlockSpec((block_t, D), lambda i: (i, 0)),
            pl.BlockSpec((ER, D), lambda i: (0, 0)),
            pl.BlockSpec((E, D), lambda i: (0, 0)),
            pl.BlockSpec((1, D), lambda i: (0, 0)),
            pl.BlockSpec((E * O, R), lambda i: (0, 0)),
            pl.BlockSpec((E, 1), lambda i: (0, 0)),
            pl.BlockSpec((1, 1), lambda i: (0, 0)),
        ],
        out_specs=pl.BlockSpec((block_t, O), lambda i: (i, 0)),
        out_shape=jax.ShapeDtypeStruct((T, O), jnp.float32),
        scratch_shapes=[pltpu.VMEM((ER, O), jnp.bfloat16)],
        compiler_params=pltpu.CompilerParams(
            dimension_semantics=("arbitrary",),
        ),
    )(inputs, w_cat, router_w, thr_w, b_flat, rb, tb)


def kernel(inputs, router_w, router_b, thr_w, thr_b, A_ws, B_ws):
    return _run(inputs, router_w, router_b, thr_w, thr_b, A_ws, B_ws)


# speedup vs baseline: 1.1748x; 1.1748x over previous
import functools

import jax
import jax.numpy as jnp
from jax.experimental import pallas as pl
from jax.experimental.pallas import tpu as pltpu

E = 8
R = 32
D = 2048
O = 2048
T = 8192
ER = E * R
NCOLS = ER + 128


NSUB = 2


def _fused_kernel(x_ref, wcat_ref, rcat_ref, bcat_ref, rb_ref, tb_ref, out_ref):
    col_e = jax.lax.broadcasted_iota(jnp.int32, (E, ER), 1) // R
    row_e = jax.lax.broadcasted_iota(jnp.int32, (E, ER), 0)
    expand = jnp.where(col_e == row_e, 1.0, 0.0).astype(jnp.bfloat16)
    sub = x_ref.shape[0] // NSUB
    for i in range(NSUB):
        x = x_ref[i * sub:(i + 1) * sub, :].astype(jnp.bfloat16)
        r_t = jax.lax.dot_general(
            rcat_ref[...], x, (((1,), (1,)), ((), ())),
            preferred_element_type=jnp.float32)
        u = jnp.exp(r_t[:E, :] + rb_ref[...])
        thr = jax.nn.sigmoid(r_t[E:E + 1, :] + tb_ref[...]) * (1.0 / E)
        v = jnp.maximum(u - thr * jnp.sum(u, axis=0, keepdims=True), 0.0)
        s = jnp.sum(v, axis=0, keepdims=True)
        s = jnp.where(s == 0.0, 1.0, s)
        w_t = (v / s).astype(jnp.bfloat16)
        h = jnp.dot(x, wcat_ref[...], preferred_element_type=jnp.float32)
        w_exp = jax.lax.dot_general(
            w_t, expand, (((0,), (0,)), ((), ())),
            preferred_element_type=jnp.float32).astype(jnp.bfloat16)
        hw = h.astype(jnp.bfloat16) * w_exp
        out_ref[i * sub:(i + 1) * sub, :] = jnp.dot(
            hw, bcat_ref[...], preferred_element_type=jnp.float32)


@functools.partial(jax.jit, static_argnames=("block_t",))
def _run(inputs, router_w, router_b, thr_w, thr_b, A_ws, B_ws, block_t=1024):
    w_cat = jnp.transpose(A_ws, (2, 0, 1)).reshape(D, ER).astype(jnp.bfloat16)
    rpad = jnp.zeros((16 - E - 1, D), dtype=jnp.float32)
    rcat = jnp.concatenate([router_w, thr_w, rpad], axis=0).astype(jnp.bfloat16)
    b_cat = jnp.transpose(B_ws, (0, 2, 1)).reshape(ER, O).astype(jnp.bfloat16)
    rb = router_b.reshape(E, 1)
    tb = thr_b.reshape(1, 1)

    grid = (T // block_t,)
    return pl.pallas_call(
        _fused_kernel,
        grid=grid,
        in_specs=[
            pl.BlockSpec((block_t, D), lambda i: (i, 0)),
            pl.BlockSpec((D, ER), lambda i: (0, 0)),
            pl.BlockSpec((16, D), lambda i: (0, 0)),
            pl.BlockSpec((ER, O), lambda i: (0, 0)),
            pl.BlockSpec((E, 1), lambda i: (0, 0)),
            pl.BlockSpec((1, 1), lambda i: (0, 0)),
        ],
        out_specs=pl.BlockSpec((block_t, O), lambda i: (i, 0)),
        out_shape=jax.ShapeDtypeStruct((T, O), jnp.float32),
        compiler_params=pltpu.CompilerParams(
            dimension_semantics=("parallel",),
        ),
    )(inputs, w_cat, rcat, b_cat, rb, tb)


def kernel(inputs, router_w, router_b, thr_w, thr_b, A_ws, B_ws):
    return _run(inputs, router_w, router_b, thr_w, thr_b, A_ws, B_ws)
